# Initial kernel scaffold; baseline (speedup 1.0000x reference)
import jax
import jax.numpy as jnp
from jax.experimental import pallas as pl


def kernel(x, ei, params):
    raise NotImplementedError("write your pallas kernel here")


# speedup vs baseline: 5.0500x; 5.0500x over previous
import functools

import jax
import jax.numpy as jnp
from jax import lax
from jax.experimental import pallas as pl
from jax.experimental.pallas import tpu as pltpu
from jax.experimental.pallas import tpu_sc as plsc

N = 10000
E = 320000
H = 128

NC = 2
NS = 16
NW = NC * NS
EPW = E // NW
C = 80
NCHUNK = EPW // C
NP = 10240
RPT = NP // NS
RCH = 128
NRCH = RPT // RCH


def _edge_body(k_hbm, q_hbm, v_hbm, src_hbm, dst_hbm, zeros_hbm, out_hbm,
               sidx, didx, kd, qs, vs, zbuf, sem1, sem2, sem3, aggsh):
    cid = lax.axis_index("c")
    sid = lax.axis_index("s")
    wid = sid * NC + cid

    row0 = sid * RPT
    for c in range(NRCH):
        pltpu.sync_copy(zeros_hbm, aggsh.at[pl.ds(row0 + c * RCH, RCH)])
    plsc.subcore_barrier()

    def chunk_body(i, carry):
        base = wid * EPW + i * C
        pltpu.sync_copy(src_hbm.at[pl.ds(base, C)], sidx)
        pltpu.sync_copy(dst_hbm.at[pl.ds(base, C)], didx)
        cp1 = pltpu.async_copy(k_hbm.at[didx], kd, sem1)
        cp2 = pltpu.async_copy(q_hbm.at[sidx], qs, sem2)
        cp3 = pltpu.async_copy(v_hbm.at[sidx], vs, sem3)
        cp1.wait()
        cp2.wait()
        cp3.wait()

        def edge_one(e, c2):
            for j in range(H // 16):
                sl = pl.ds(j * 16, 16)
                kk = kd[e, sl]
                qq = qs[e, sl]
                vv = vs[e, sl]
                eta = 1.0 / (1.0 + jnp.exp(-(kk + qq)))
                vs[e, sl] = eta * vv
            return c2

        lax.fori_loop(0, C, edge_one, 0, unroll=False)
        pltpu.sync_copy(vs, aggsh.at[didx], add=True)
        return carry

    lax.fori_loop(0, NCHUNK, chunk_body, 0, unroll=False)
    plsc.subcore_barrier()

    for c in range(NRCH):
        r = row0 + c * RCH
        pltpu.sync_copy(aggsh.at[pl.ds(r, RCH)], zbuf)
        pltpu.sync_copy(zbuf, out_hbm.at[cid, pl.ds(r, RCH)])


@jax.jit
def _edge_pass(k, q, v, src, dst, zeros):
    mesh = plsc.VectorSubcoreMesh(core_axis_name="c", subcore_axis_name="s")
    f = pl.kernel(
        _edge_body,
        out_type=jax.ShapeDtypeStruct((NC, NP, H), jnp.float32),
        mesh=mesh,
        scratch_types=[
            pltpu.VMEM((C,), jnp.int32),
            pltpu.VMEM((C,), jnp.int32),
            pltpu.VMEM((C, H), jnp.float32),
            pltpu.VMEM((C, H), jnp.float32),
            pltpu.VMEM((C, H), jnp.float32),
            pltpu.VMEM((RCH, H), jnp.float32),
            pltpu.SemaphoreType.DMA,
            pltpu.SemaphoreType.DMA,
            pltpu.SemaphoreType.DMA,
            pltpu.VMEM_SHARED((NP, H), jnp.float32),
        ],
    )
    return f(k, q, v, src, dst, zeros)


BLK = 2000


def _dense0_body(x_ref, wp_ref, bp_ref, wc_ref, bc_ref, out_ref):
    h = jnp.maximum(jnp.dot(x_ref[...], wp_ref[...],
                            preferred_element_type=jnp.float32)
                    + bp_ref[...], 0.0)
    out_ref[...] = jnp.dot(h, wc_ref[...],
                           preferred_element_type=jnp.float32) + bc_ref[...]


@jax.jit
def _dense0(x, wp, bp, wc, bc):
    m = wc.shape[1]
    return pl.pallas_call(
        _dense0_body,
        grid=(N // BLK,),
        in_specs=[
            pl.BlockSpec((BLK, H), lambda i: (i, 0)),
            pl.BlockSpec((H, H), lambda i: (0, 0)),
            pl.BlockSpec((1, H), lambda i: (0, 0)),
            pl.BlockSpec((H, m), lambda i: (0, 0)),
            pl.BlockSpec((1, m), lambda i: (0, 0)),
        ],
        out_specs=pl.BlockSpec((BLK, m), lambda i: (i, 0)),
        out_shape=jax.ShapeDtypeStruct((N, m), jnp.float32),
    )(x, wp, bp, wc, bc)


def _stats_body(a0_ref, a1_ref, s_ref, pre_ref, sum_ref, sq_ref):
    i = pl.program_id(0)
    pre = a0_ref[...] + a1_ref[...] + s_ref[...]
    pre_ref[...] = pre
    bs = jnp.sum(pre, axis=0, keepdims=True)
    bq = jnp.sum(pre * pre, axis=0, keepdims=True)

    @pl.when(i == 0)
    def _():
        sum_ref[...] = bs
        sq_ref[...] = bq

    @pl.when(i > 0)
    def _():
        sum_ref[...] += bs
        sq_ref[...] += bq


@jax.jit
def _stats(a0, a1, s):
    return pl.pallas_call(
        _stats_body,
        grid=(N // BLK,),
        in_specs=[pl.BlockSpec((BLK, H), lambda i: (i, 0))] * 3,
        out_specs=[
            pl.BlockSpec((BLK, H), lambda i: (i, 0)),
            pl.BlockSpec((1, H), lambda i: (0, 0)),
            pl.BlockSpec((1, H), lambda i: (0, 0)),
        ],
        out_shape=[
            jax.ShapeDtypeStruct((N, H), jnp.float32),
            jax.ShapeDtypeStruct((1, H), jnp.float32),
            jax.ShapeDtypeStruct((1, H), jnp.float32),
        ],
    )(a0, a1, s)


def _normproj_body(pre_ref, sum_ref, sq_ref, g_ref, be_ref, wc_ref, bc_ref,
                   out_ref):
    mu = sum_ref[...] / N
    var = sq_ref[...] / N - mu * mu
    scale = g_ref[...] * lax.rsqrt(var + 1e-5)
    h = jnp.maximum((pre_ref[...] - mu) * scale + be_ref[...], 0.0)
    out_ref[...] = jnp.dot(h, wc_ref[...],
                           preferred_element_type=jnp.float32) + bc_ref[...]


@jax.jit
def _normproj(pre, sm, sq, g, be, wc, bc):
    m = wc.shape[1]
    return pl.pallas_call(
        _normproj_body,
        grid=(N // BLK,),
        in_specs=[
            pl.BlockSpec((BLK, H), lambda i: (i, 0)),
            pl.BlockSpec((1, H), lambda i: (0, 0)),
            pl.BlockSpec((1, H), lambda i: (0, 0)),
            pl.BlockSpec((1, H), lambda i: (0, 0)),
            pl.BlockSpec((1, H), lambda i: (0, 0)),
            pl.BlockSpec((H, m), lambda i: (0, 0)),
            pl.BlockSpec((1, m), lambda i: (0, 0)),
        ],
        out_specs=pl.BlockSpec((BLK, m), lambda i: (i, 0)),
        out_shape=jax.ShapeDtypeStruct((N, m), jnp.float32),
    )(pre, sm, sq, g, be, wc, bc)


def _wcat(c):
    wc = jnp.concatenate([c['Wk'], c['Wq'], c['Wv'], c['Ws']], axis=1)
    bc = jnp.concatenate([c['bk'], c['bq'], c['bv'], c['b']])[None, :]
    return wc, bc


def kernel(x, ei, params):
    p = params
    zeros = jnp.zeros((RCH, H), jnp.float32)

    wc1, bc1 = _wcat(p['c1'])
    proj = _dense0(x, p['Wp'], p['bp'][None, :], wc1, bc1)

    for i in (1, 2, 3):
        k = proj[:, 0:H]
        q = proj[:, H:2 * H]
        v = proj[:, 2 * H:3 * H]
        s = proj[:, 3 * H:4 * H]
        aggp = _edge_pass(k, q, v, ei[0], ei[1], zeros)
        pre, sm, sq = _stats(aggp[0, :N], aggp[1, :N], s)
        if i < 3:
            wc, bc = _wcat(p['c%d' % (i + 1)])
        else:
            wc, bc = p['Wh'], p['bh'][None, :]
        proj = _normproj(pre, sm, sq, p['g%d' % i][None, :],
                         p['be%d' % i][None, :], wc, bc)
    return proj
